# baseline (speedup 1.0000x reference)
import functools

import jax
import jax.numpy as jnp
from jax import lax
from jax.experimental import pallas as pl
from jax.experimental.pallas import tpu as pltpu
from jax.experimental.pallas import tpu_sc as plsc

EMB_DIM = 64
HIDDEN = 128
LANES = 16


def _repack_body(a1_ref, a2_ref, eye_ref, outa_ref):
    eye = eye_ref[...]

    def t(ref):
        return lax.dot_general(ref[...], eye, (((0,), (0,)), ((), ())),
                               preferred_element_type=jnp.float32)

    outa_ref[:, :EMB_DIM] = t(a1_ref)
    outa_ref[:, EMB_DIM:] = t(a2_ref)


_RC = 4608


@functools.lru_cache(maxsize=None)
def _make_repack(V: int, H: int):
    hb = H // _RC
    spec_lo = pl.BlockSpec((EMB_DIM, _RC), lambda i: (0, i))
    spec_hi = pl.BlockSpec((EMB_DIM, _RC), lambda i: (0, i + hb))
    out_spec = pl.BlockSpec((_RC, 2 * EMB_DIM), lambda i: (i, 0))
    out_ty = jax.ShapeDtypeStruct((H, 2 * EMB_DIM), jnp.float32)
    return pl.pallas_call(
        _repack_body,
        compiler_params=pltpu.CompilerParams(
            dimension_semantics=("parallel",)),
        grid_spec=pl.GridSpec(
            grid=(hb,),
            in_specs=[
                spec_lo, spec_hi,
                pl.BlockSpec((EMB_DIM, EMB_DIM), lambda i: (0, 0)),
            ],
            out_specs=out_spec,
        ),
        out_shape=out_ty,
    )


def _split_point(V: int) -> int:
    H = ((V + 1) // 2 + _RC - 1) // _RC * _RC
    assert 2 * H <= _RC * ((V + _RC - 1) // _RC), (V, H)
    return H


def _repack(ta_t):
    V = ta_t.shape[1]
    H = _split_point(V)
    eye = jnp.eye(EMB_DIM, dtype=jnp.float32)
    return _make_repack(V, H)(ta_t, ta_t, eye)


@functools.lru_cache(maxsize=None)
def _make_sc_gather(B: int, H: int):
    info = plsc.get_sparse_core_info()
    NC, NS = info.num_cores, info.num_subcores
    NW = NC * NS
    b_per_w = B // NW
    CH = 256
    n_ch = b_per_w // CH
    assert B % (CH * NW) == 0
    D2 = 2 * EMB_DIM

    mesh = plsc.VectorSubcoreMesh(core_axis_name="c", subcore_axis_name="s")

    @functools.partial(
        pl.kernel,
        mesh=mesh,
        out_type=jax.ShapeDtypeStruct((B, D2), jnp.float32),
        scratch_types=[
            pltpu.VMEM((b_per_w,), jnp.int32),
            pltpu.VMEM((CH, D2), jnp.float32),
            pltpu.VMEM((CH, D2), jnp.float32),
            pltpu.SemaphoreType.DMA,
            pltpu.SemaphoreType.DMA,
        ],
    )
    def gather_k(tp_hbm, idx_hbm, out_hbm, idx_v, rows0_v, rows1_v,
                 sem0, sem1):
        wid = lax.axis_index("s") * NC + lax.axis_index("c")
        base = wid * b_per_w
        pltpu.sync_copy(idx_hbm.at[pl.ds(base, b_per_w)], idx_v)

        def fold(v, carry):
            sl = pl.ds(v * LANES, LANES)
            a = idx_v[sl]
            idx_v[sl] = jnp.where(a < H, a, a - H)
            return carry

        lax.fori_loop(0, b_per_w // LANES, fold, 0)

        def chunk(c, carry):
            off = c * 2 * CH
            c0 = pltpu.async_copy(
                tp_hbm.at[idx_v.at[pl.ds(off, CH)]], rows0_v, sem0)
            c1 = pltpu.async_copy(
                tp_hbm.at[idx_v.at[pl.ds(off + CH, CH)]], rows1_v, sem1)
            c0.wait()
            c1.wait()
            cbase = base + off
            pltpu.sync_copy(rows0_v, out_hbm.at[pl.ds(cbase, CH)])
            pltpu.sync_copy(rows1_v, out_hbm.at[pl.ds(cbase + CH, CH)])
            return carry

        lax.fori_loop(0, n_ch // 2, chunk, 0)

    return gather_k


def _mlp_body(dvh_ref, ovh_ref, p0_ref, p1_ref, g_ref, bt_ref, w1_ref, b1_ref,
              w2_ref, b2_ref, w3_ref, b3_ref, out_ref):
    dvh = dvh_ref[...]
    ovh = ovh_ref[...]
    p0 = p0_ref[...] == 1
    p1 = p1_ref[...] == 1
    dv = jnp.where(p0, dvh[:, EMB_DIM:], dvh[:, :EMB_DIM])
    ov = jnp.where(p1, ovh[:, EMB_DIM:], ovh[:, :EMB_DIM])
    n = 2 * EMB_DIM
    mean = (jnp.sum(dv, axis=1, keepdims=True)
            + jnp.sum(ov, axis=1, keepdims=True)) / n
    dvc = dv - mean
    ovc = ov - mean
    var = (jnp.sum(dvc * dvc, axis=1, keepdims=True)
           + jnp.sum(ovc * ovc, axis=1, keepdims=True)) / n
    inv = lax.rsqrt(var + 1e-5)
    g = g_ref[...]
    bt = bt_ref[...]
    hd = dvc * inv * g[:, :EMB_DIM] + bt[:, :EMB_DIM]
    ho = ovc * inv * g[:, EMB_DIM:] + bt[:, EMB_DIM:]
    w1 = w1_ref[...]
    h1 = (jnp.dot(hd, w1[:EMB_DIM, :], preferred_element_type=jnp.float32)
          + jnp.dot(ho, w1[EMB_DIM:, :], preferred_element_type=jnp.float32)
          + b1_ref[...])
    h1 = jnp.maximum(h1, 0.0)
    h2 = jnp.dot(h1, w2_ref[...], preferred_element_type=jnp.float32) + b2_ref[...]
    h2 = jnp.maximum(h2, 0.0)
    y = jnp.dot(h2, w3_ref[...], preferred_element_type=jnp.float32) + b3_ref[...]
    out_ref[...] = jnp.transpose(jax.nn.sigmoid(y), (1, 0))


@functools.lru_cache(maxsize=None)
def _make_tc_mlp(B: int, BB: int):
    full = lambda i: (0, 0)
    grid_spec = pl.GridSpec(
        grid=(B // BB,),
        in_specs=[
            pl.BlockSpec((BB, 2 * EMB_DIM), lambda i: (i, 0)),
            pl.BlockSpec((BB, 2 * EMB_DIM), lambda i: (i, 0)),
            pl.BlockSpec((BB, 1), lambda i: (i, 0)),
            pl.BlockSpec((BB, 1), lambda i: (i, 0)),
            pl.BlockSpec((1, 2 * EMB_DIM), full),
            pl.BlockSpec((1, 2 * EMB_DIM), full),
            pl.BlockSpec((2 * EMB_DIM, HIDDEN), full),
            pl.BlockSpec((1, HIDDEN), full),
            pl.BlockSpec((HIDDEN, HIDDEN // 2), full),
            pl.BlockSpec((1, HIDDEN // 2), full),
            pl.BlockSpec((HIDDEN // 2, 2), full),
            pl.BlockSpec((1, 2), full),
        ],
        out_specs=pl.BlockSpec((2, BB), lambda i: (0, i)),
    )
    return pl.pallas_call(
        _mlp_body,
        grid_spec=grid_spec,
        out_shape=jax.ShapeDtypeStruct((2, B), jnp.float32),
    )


def kernel(x_idx, dv_table, ov_table, ln_gamma, ln_beta, W1, b1, W2, b2, W3, b3):
    B = x_idx.shape[0]
    idx0 = x_idx[:, 0].astype(jnp.int32)
    idx1 = x_idx[:, 1].astype(jnp.int32)
    H = _split_point(dv_table.shape[0])
    gather = _make_sc_gather(B, H)
    dvp = _repack(dv_table.T)
    dvh = gather(dvp, idx0)
    ovp = _repack(ov_table.T)
    ovh = gather(ovp, idx1)
    mlp = _make_tc_mlp(B, 8192)
    out_t = mlp(dvh, ovh,
                (idx0 >= H).astype(jnp.int32).reshape(-1, 1),
                (idx1 >= H).astype(jnp.int32).reshape(-1, 1),
                ln_gamma.reshape(1, -1), ln_beta.reshape(1, -1),
                W1, b1.reshape(1, -1), W2, b2.reshape(1, -1),
                W3, b3.reshape(1, -1))
    return out_t.T


# speedup vs baseline: 1.4160x; 1.4160x over previous
import functools

import jax
import jax.numpy as jnp
from jax import lax
from jax.experimental import pallas as pl
from jax.experimental.pallas import tpu as pltpu
from jax.experimental.pallas import tpu_sc as plsc

EMB_DIM = 64
HIDDEN = 128
LANES = 16


def _repack_body(a1_ref, a2_ref, b1_ref, b2_ref, sel_ref, outa_ref, outb_ref):
    sel = sel_ref[...]

    def t(lo_ref, hi_ref):
        stacked = jnp.concatenate([lo_ref[...], hi_ref[...]], axis=0)
        return lax.dot_general(stacked, sel, (((0,), (0,)), ((), ())),
                               preferred_element_type=jnp.float32)

    outa_ref[...] = t(a1_ref, a2_ref)
    outb_ref[...] = t(b1_ref, b2_ref)


_RC = 4608


@functools.lru_cache(maxsize=None)
def _make_repack(V: int, H: int):
    hb = H // _RC
    spec_lo = pl.BlockSpec((EMB_DIM, _RC), lambda i: (0, i))
    spec_hi = pl.BlockSpec((EMB_DIM, _RC), lambda i: (0, i + hb))
    out_spec = pl.BlockSpec((_RC, 2 * EMB_DIM), lambda i: (i, 0))
    out_ty = jax.ShapeDtypeStruct((H, 2 * EMB_DIM), jnp.float32)
    return pl.pallas_call(
        _repack_body,
        compiler_params=pltpu.CompilerParams(
            dimension_semantics=("parallel",)),
        grid_spec=pl.GridSpec(
            grid=(hb,),
            in_specs=[
                spec_lo, spec_hi,
                pl.BlockSpec((EMB_DIM, _RC), lambda i: (0, i)),
                pl.BlockSpec((EMB_DIM, _RC), lambda i: (0, i + hb)),
                pl.BlockSpec((2 * EMB_DIM, 2 * EMB_DIM), lambda i: (0, 0)),
            ],
            out_specs=[out_spec,
                       pl.BlockSpec((_RC, 2 * EMB_DIM), lambda i: (i, 0))],
        ),
        out_shape=[out_ty, out_ty],
    )


def _split_point(V: int) -> int:
    H = ((V + 1) // 2 + _RC - 1) // _RC * _RC
    assert 2 * H <= _RC * ((V + _RC - 1) // _RC), (V, H)
    return H


def _repack2(ta_t, tb_t):
    V = ta_t.shape[1]
    H = _split_point(V)
    sel = jnp.eye(2 * EMB_DIM, dtype=jnp.float32)
    return _make_repack(V, H)(ta_t, ta_t, tb_t, tb_t, sel)


@functools.lru_cache(maxsize=None)
def _make_sc_gather(B: int, H: int):
    info = plsc.get_sparse_core_info()
    NC, NS = info.num_cores, info.num_subcores
    NW = NC * NS
    b_per_w = B // NW
    CH = 256
    n_ch = b_per_w // CH
    assert B % (CH * NW) == 0
    D2 = 2 * EMB_DIM

    mesh = plsc.VectorSubcoreMesh(core_axis_name="c", subcore_axis_name="s")

    @functools.partial(
        pl.kernel,
        mesh=mesh,
        out_type=[
            jax.ShapeDtypeStruct((B, D2), jnp.float32),
            jax.ShapeDtypeStruct((B, D2), jnp.float32),
        ],
        scratch_types=[
            pltpu.VMEM((b_per_w,), jnp.int32),
            pltpu.VMEM((b_per_w,), jnp.int32),
            pltpu.VMEM((CH, D2), jnp.float32),
            pltpu.VMEM((CH, D2), jnp.float32),
            pltpu.SemaphoreType.DMA,
            pltpu.SemaphoreType.DMA,
        ],
    )
    def gather_k(dvp_hbm, ovp_hbm, pidx0_hbm, pidx1_hbm, dv_out, ov_out,
                 idx0_v, idx1_v, rows0_v, rows1_v, sem0, sem1):
        wid = lax.axis_index("s") * NC + lax.axis_index("c")
        base = wid * b_per_w
        pltpu.sync_copy(pidx0_hbm.at[pl.ds(base, b_per_w)], idx0_v)
        pltpu.sync_copy(pidx1_hbm.at[pl.ds(base, b_per_w)], idx1_v)

        def fold(v, carry):
            sl = pl.ds(v * LANES, LANES)
            a = idx0_v[sl]
            idx0_v[sl] = jnp.where(a < H, a, a - H)
            b = idx1_v[sl]
            idx1_v[sl] = jnp.where(b < H, b, b - H)
            return carry

        lax.fori_loop(0, b_per_w // LANES, fold, 0)

        def chunk(c, carry):
            off = c * CH
            c0 = pltpu.async_copy(
                dvp_hbm.at[idx0_v.at[pl.ds(off, CH)]], rows0_v, sem0)
            c1 = pltpu.async_copy(
                ovp_hbm.at[idx1_v.at[pl.ds(off, CH)]], rows1_v, sem1)
            c0.wait()
            c1.wait()
            cbase = base + off
            pltpu.sync_copy(rows0_v, dv_out.at[pl.ds(cbase, CH)])
            pltpu.sync_copy(rows1_v, ov_out.at[pl.ds(cbase, CH)])
            return carry

        lax.fori_loop(0, n_ch, chunk, 0)

    return gather_k


def _mlp_body(dvh_ref, ovh_ref, p0_ref, p1_ref, g_ref, bt_ref, w1_ref, b1_ref,
              w2_ref, b2_ref, w3_ref, b3_ref, out_ref):
    dvh = dvh_ref[...]
    ovh = ovh_ref[...]
    p0 = p0_ref[...] == 1
    p1 = p1_ref[...] == 1
    dv = jnp.where(p0, dvh[:, EMB_DIM:], dvh[:, :EMB_DIM])
    ov = jnp.where(p1, ovh[:, EMB_DIM:], ovh[:, :EMB_DIM])
    n = 2 * EMB_DIM
    mean = (jnp.sum(dv, axis=1, keepdims=True)
            + jnp.sum(ov, axis=1, keepdims=True)) / n
    dvc = dv - mean
    ovc = ov - mean
    var = (jnp.sum(dvc * dvc, axis=1, keepdims=True)
           + jnp.sum(ovc * ovc, axis=1, keepdims=True)) / n
    inv = lax.rsqrt(var + 1e-5)
    g = g_ref[...]
    bt = bt_ref[...]
    hd = dvc * inv * g[:, :EMB_DIM] + bt[:, :EMB_DIM]
    ho = ovc * inv * g[:, EMB_DIM:] + bt[:, EMB_DIM:]
    w1 = w1_ref[...]
    h1 = (jnp.dot(hd, w1[:EMB_DIM, :], preferred_element_type=jnp.float32)
          + jnp.dot(ho, w1[EMB_DIM:, :], preferred_element_type=jnp.float32)
          + b1_ref[...])
    h1 = jnp.maximum(h1, 0.0)
    h2 = jnp.dot(h1, w2_ref[...], preferred_element_type=jnp.float32) + b2_ref[...]
    h2 = jnp.maximum(h2, 0.0)
    y = jnp.dot(h2, w3_ref[...], preferred_element_type=jnp.float32) + b3_ref[...]
    out_ref[...] = jnp.transpose(jax.nn.sigmoid(y), (1, 0))


@functools.lru_cache(maxsize=None)
def _make_tc_mlp(B: int, BB: int):
    full = lambda i: (0, 0)
    grid_spec = pl.GridSpec(
        grid=(B // BB,),
        in_specs=[
            pl.BlockSpec((BB, 2 * EMB_DIM), lambda i: (i, 0)),
            pl.BlockSpec((BB, 2 * EMB_DIM), lambda i: (i, 0)),
            pl.BlockSpec((BB, 1), lambda i: (i, 0)),
            pl.BlockSpec((BB, 1), lambda i: (i, 0)),
            pl.BlockSpec((1, 2 * EMB_DIM), full),
            pl.BlockSpec((1, 2 * EMB_DIM), full),
            pl.BlockSpec((2 * EMB_DIM, HIDDEN), full),
            pl.BlockSpec((1, HIDDEN), full),
            pl.BlockSpec((HIDDEN, HIDDEN // 2), full),
            pl.BlockSpec((1, HIDDEN // 2), full),
            pl.BlockSpec((HIDDEN // 2, 2), full),
            pl.BlockSpec((1, 2), full),
        ],
        out_specs=pl.BlockSpec((2, BB), lambda i: (0, i)),
    )
    return pl.pallas_call(
        _mlp_body,
        grid_spec=grid_spec,
        out_shape=jax.ShapeDtypeStruct((2, B), jnp.float32),
    )


def kernel(x_idx, dv_table, ov_table, ln_gamma, ln_beta, W1, b1, W2, b2, W3, b3):
    B = x_idx.shape[0]
    idx0 = x_idx[:, 0].astype(jnp.int32)
    idx1 = x_idx[:, 1].astype(jnp.int32)
    H = _split_point(dv_table.shape[0])
    dvp, ovp = _repack2(dv_table.T, ov_table.T)
    dvh, ovh = _make_sc_gather(B, H)(dvp, ovp, idx0, idx1)
    mlp = _make_tc_mlp(B, 8192)
    out_t = mlp(dvh, ovh,
                (idx0 >= H).astype(jnp.int32).reshape(-1, 1),
                (idx1 >= H).astype(jnp.int32).reshape(-1, 1),
                ln_gamma.reshape(1, -1), ln_beta.reshape(1, -1),
                W1, b1.reshape(1, -1), W2, b2.reshape(1, -1),
                W3, b3.reshape(1, -1))
    return out_t.T
